# baseline (speedup 1.0000x reference)
import functools

import jax
import jax.numpy as jnp
from jax import lax
from jax.experimental import pallas as pl
from jax.experimental.pallas import tpu as pltpu
from jax.experimental.pallas import tpu_sc as plsc

N = 10000
K = 16
NP = 10240
ROW_BLK = 256
DENSE_BLK = 512
STAT_BLK = 400
EPS = 1e-5
NW = 32


def _knn_body(posr_ref, posT_ref, idx_ref):
    r0 = posr_ref[:, 0:1]
    r1 = posr_ref[:, 1:2]
    r2 = posr_ref[:, 2:3]
    c0 = posT_ref[0:1, :]
    c1 = posT_ref[1:2, :]
    c2 = posT_ref[2:3, :]
    d = -(((r0 - c0) ** 2 + (r1 - c1) ** 2) + (r2 - c2) ** 2)
    h = NP // 2
    dL = d[:, :h]
    dR = d[:, h:]
    jL = lax.broadcasted_iota(jnp.int32, (ROW_BLK, h), 1).astype(jnp.float32)
    jR = jL + float(h)
    win = dL >= dR
    val = jnp.where(win, dL, dR)
    vidx = jnp.where(win, jL, jR)
    lose = jnp.where(win, dR, dL)
    loseidx = jnp.where(win, jR, jL)
    bigf = jnp.float32(3.0e7)
    neginf = jnp.float32(-jnp.inf)
    cols = []
    m = jnp.max(val, axis=1, keepdims=True)
    for t in range(K):
        js = jnp.min(jnp.where(val == m, vidx, bigf), axis=1, keepdims=True)
        cols.append(js)
        jm = jnp.where(js >= h, js - float(h), js)
        smask = jL == jm
        val = jnp.where(smask, lose, val)
        vidx = jnp.where(smask, loseidx, vidx)
        lose = jnp.where(smask, neginf, lose)
        if t < K - 1:
            m = jnp.max(val, axis=1, keepdims=True)
    idx_ref[...] = jnp.concatenate(cols, axis=1).astype(jnp.int32)


def _knn(pos_pad, posT, half):
    nb = NP // ROW_BLK // 2
    return pl.pallas_call(
        _knn_body,
        grid=(nb,),
        in_specs=[
            pl.BlockSpec((ROW_BLK, 8), lambda i: (i + half * nb, 0)),
            pl.BlockSpec((8, NP), lambda i: (0, 0)),
        ],
        out_specs=pl.BlockSpec((ROW_BLK, K), lambda i: (i, 0)),
        out_shape=jax.ShapeDtypeStruct((NP // 2, K), jnp.int32),
    )(pos_pad, posT)


def _dense1_body(x_ref, W0_ref, b0_ref, Wa_ref, Wb_ref, be_ref,
                 f1_ref, t1_ref, s1_ref):
    x = x_ref[...]
    f1 = jnp.dot(x, W0_ref[...], preferred_element_type=jnp.float32) + b0_ref[...]
    f1_ref[...] = f1
    t1 = jnp.dot(f1, Wb_ref[...], preferred_element_type=jnp.float32)
    t1_ref[...] = jnp.concatenate([t1, jnp.zeros_like(t1)], axis=1)
    s1_ref[...] = (jnp.dot(f1, Wa_ref[...], preferred_element_type=jnp.float32)
                   - t1 + be_ref[...])


def _dense1(x_pad, W0, b0, Wa1, Wb1, be1):
    g = NP // DENSE_BLK
    blk = lambda r, c: pl.BlockSpec((r, c), lambda i: (i, 0))
    full = lambda r, c: pl.BlockSpec((r, c), lambda i: (0, 0))
    return pl.pallas_call(
        _dense1_body,
        grid=(g,),
        in_specs=[blk(DENSE_BLK, 64), full(64, 64), full(1, 64),
                  full(64, 64), full(64, 64), full(1, 64)],
        out_specs=[blk(DENSE_BLK, 64), blk(DENSE_BLK, 128), blk(DENSE_BLK, 64)],
        out_shape=[jax.ShapeDtypeStruct((NP, 64), jnp.float32),
                   jax.ShapeDtypeStruct((NP, 128), jnp.float32),
                   jax.ShapeDtypeStruct((NP, 64), jnp.float32)],
    )(x_pad, W0, b0, Wa1, Wb1, be1)


def _sc_gather_reduce(t_pad, idx_flat, C):
    G = 8
    W128 = 128
    NN = NP // 2
    npw = NN // NW
    nch = npw // G
    gk = G * K
    mesh = plsc.VectorSubcoreMesh(core_axis_name="c", subcore_axis_name="s")

    @functools.partial(
        pl.kernel,
        out_type=jax.ShapeDtypeStruct((4, NN, W128), jnp.float32),
        mesh=mesh,
        scratch_types=[
            pltpu.VMEM((npw * K,), jnp.int32),
            pltpu.VMEM((gk, W128), jnp.float32),
            pltpu.VMEM((4, G, W128), jnp.float32),
            pltpu.SemaphoreType.DMA,
        ],
    )
    def k(t_hbm, idx_hbm, out_hbm, idx_v, rows_v, stage_v, sem):
        wid = lax.axis_index("s") * 2 + lax.axis_index("c")
        base0 = wid * npw
        pltpu.sync_copy(idx_hbm.at[pl.ds(base0 * K, npw * K)], idx_v)

        def chunk(ch, carry):
            base = base0 + ch * G
            pltpu.async_copy(
                t_hbm.at[idx_v.at[pl.ds(ch * gk, gk)]], rows_v, sem).wait()
            for n in range(G):
                for cg in range(C // 16):
                    sl = pl.ds(cg * 16, 16)
                    r = rows_v[n * K, sl]
                    u = r
                    v = r * r
                    mx = r
                    mn = r
                    for kk in range(1, K):
                        r = rows_v[n * K + kk, sl]
                        u = u + r
                        v = v + r * r
                        mx = jnp.maximum(mx, r)
                        mn = jnp.minimum(mn, r)
                    stage_v[0, n, sl] = u
                    stage_v[1, n, sl] = v
                    stage_v[2, n, sl] = mx
                    stage_v[3, n, sl] = mn
            pltpu.sync_copy(stage_v, out_hbm.at[:, pl.ds(base, G), :])
            return carry

        lax.fori_loop(0, nch, chunk, 0)

    out = k(t_pad, idx_flat)
    return out[0], out[1], out[2], out[3]


def _stats_body(s_ref, u_ref, v_ref, o_ref):
    @pl.when(pl.program_id(0) == 0)
    def _():
        o_ref[...] = jnp.zeros_like(o_ref)

    s = s_ref[...]
    c = s.shape[1]
    u = u_ref[:, :c]
    v = v_ref[:, :c]
    acc = jnp.concatenate([
        jnp.sum(s, axis=0, keepdims=True),
        jnp.sum(s * s, axis=0, keepdims=True),
        jnp.sum(u, axis=0, keepdims=True),
        jnp.sum(s * u, axis=0, keepdims=True),
        jnp.sum(v, axis=0, keepdims=True),
        jnp.zeros((3, s.shape[1]), jnp.float32),
    ], axis=0)
    o_ref[...] += acc


def _stats(s, u, v, C):
    g = N // STAT_BLK
    blk = pl.BlockSpec((STAT_BLK, C), lambda i: (i, 0))
    blk128 = pl.BlockSpec((STAT_BLK, 128), lambda i: (i, 0))
    return pl.pallas_call(
        _stats_body,
        grid=(g,),
        in_specs=[blk, blk128, blk128],
        out_specs=pl.BlockSpec((8, C), lambda i: (0, 0)),
        out_shape=jax.ShapeDtypeStruct((8, C), jnp.float32),
    )(s, u, v)


def _bn_scale_shift(S_ref, g_ref, bt_ref):
    nk = float(N * K)
    S = S_ref[...]
    mu = (K * S[0:1, :] + S[2:3, :]) / nk
    ex2 = (K * S[1:2, :] + 2.0 * S[3:4, :] + S[4:5, :]) / nk
    var = ex2 - mu * mu
    scale = g_ref[...] / jnp.sqrt(var + EPS)
    shift = bt_ref[...] - mu * scale
    return scale, shift


def _dense2_body(S_ref, g_ref, bt_ref, f1_ref, s1_ref, mx_ref, mn_ref,
                 W2_ref, b2_ref, Wa3_ref, Wb3_ref, be3_ref,
                 f2_ref, f3_ref, t3_ref, s3_ref):
    scale, shift = _bn_scale_shift(S_ref, g_ref, bt_ref)
    sel = jnp.where(scale >= 0.0, mx_ref[:, :64], mn_ref[:, :64])
    f2 = jnp.maximum(scale * (s1_ref[...] + sel) + shift, 0.0) + f1_ref[...]
    f2_ref[...] = f2
    f3 = jnp.dot(f2, W2_ref[...], preferred_element_type=jnp.float32) + b2_ref[...]
    f3_ref[...] = f3
    t3 = jnp.dot(f3, Wb3_ref[...], preferred_element_type=jnp.float32)
    t3_ref[...] = t3
    s3_ref[...] = (jnp.dot(f3, Wa3_ref[...], preferred_element_type=jnp.float32)
                   - t3 + be3_ref[...])


def _dense2(S1, g1, bt1, f1, s1, mx1, mn1, W2, b2, Wa3, Wb3, be3):
    g = NP // DENSE_BLK
    blk = lambda r, c: pl.BlockSpec((r, c), lambda i: (i, 0))
    full = lambda r, c: pl.BlockSpec((r, c), lambda i: (0, 0))
    return pl.pallas_call(
        _dense2_body,
        grid=(g,),
        in_specs=[full(8, 64), full(1, 64), full(1, 64),
                  blk(DENSE_BLK, 64), blk(DENSE_BLK, 64),
                  blk(DENSE_BLK, 128), blk(DENSE_BLK, 128),
                  full(64, 128), full(1, 128),
                  full(128, 128), full(128, 128), full(1, 128)],
        out_specs=[blk(DENSE_BLK, 64), blk(DENSE_BLK, 128),
                   blk(DENSE_BLK, 128), blk(DENSE_BLK, 128)],
        out_shape=[jax.ShapeDtypeStruct((NP, 64), jnp.float32),
                   jax.ShapeDtypeStruct((NP, 128), jnp.float32),
                   jax.ShapeDtypeStruct((NP, 128), jnp.float32),
                   jax.ShapeDtypeStruct((NP, 128), jnp.float32)],
    )(S1, g1, bt1, f1, s1, mx1, mn1, W2, b2, Wa3, Wb3, be3)


def _final_partial_body(f1_ref, f2_ref, f3_ref,
                        Wo1_ref, Wo2_ref, Wo3_ref, bout_ref, o_ref):
    o = jnp.dot(f1_ref[...], Wo1_ref[...], preferred_element_type=jnp.float32)
    o = o + jnp.dot(f2_ref[...], Wo2_ref[...], preferred_element_type=jnp.float32)
    o = o + jnp.dot(f3_ref[...], Wo3_ref[...], preferred_element_type=jnp.float32)
    o_ref[...] = o + bout_ref[...]


def _final_partial(f1, f2, f3, Wo1, Wo2, Wo3, bout):
    g = NP // DENSE_BLK
    blk = lambda r, c: pl.BlockSpec((r, c), lambda i: (i, 0))
    full = lambda r, c: pl.BlockSpec((r, c), lambda i: (0, 0))
    return pl.pallas_call(
        _final_partial_body,
        grid=(g,),
        in_specs=[blk(DENSE_BLK, 64), blk(DENSE_BLK, 64),
                  blk(DENSE_BLK, 128),
                  full(64, 128), full(64, 128),
                  full(128, 128), full(1, 128)],
        out_specs=blk(DENSE_BLK, 128),
        out_shape=jax.ShapeDtypeStruct((NP, 128), jnp.float32),
    )(f1, f2, f3, Wo1, Wo2, Wo3, bout)


def _final2_body(S_ref, g_ref, bt_ref, f3_ref, s3_ref, mx_ref, mn_ref,
                 part_ref, Wo4_ref, o_ref):
    scale, shift = _bn_scale_shift(S_ref, g_ref, bt_ref)
    sel = jnp.where(scale >= 0.0, mx_ref[...], mn_ref[...])
    f4 = jnp.maximum(scale * (s3_ref[...] + sel) + shift, 0.0) + f3_ref[...]
    o_ref[...] = part_ref[...] + jnp.dot(
        f4, Wo4_ref[...], preferred_element_type=jnp.float32)


def _final2(S3, g3, bt3, f3, s3, mx3, mn3, partial, Wo4):
    g = NP // DENSE_BLK
    blk = lambda r, c: pl.BlockSpec((r, c), lambda i: (i, 0))
    full = lambda r, c: pl.BlockSpec((r, c), lambda i: (0, 0))
    return pl.pallas_call(
        _final2_body,
        grid=(g,),
        in_specs=[full(8, 128), full(1, 128), full(1, 128),
                  blk(DENSE_BLK, 128), blk(DENSE_BLK, 128),
                  blk(DENSE_BLK, 128), blk(DENSE_BLK, 128),
                  blk(DENSE_BLK, 128), full(128, 128)],
        out_specs=blk(DENSE_BLK, 128),
        out_shape=jax.ShapeDtypeStruct((NP, 128), jnp.float32),
    )(S3, g3, bt3, f3, s3, mx3, mn3, partial, Wo4)


def kernel(point_features, point_coords, W0, b0, We1, be1, g1, bt1,
           W2, b2, We3, be3, g3, bt3, Wout, bout):
    pos = point_coords[:, 1:4]
    x = jnp.concatenate([pos, point_features], axis=1)
    x_pad = jnp.pad(x, ((0, NP - N), (0, 0)))
    pos_pad = jnp.pad(pos, ((0, NP - N), (0, 0)), constant_values=1e6)
    pos_pad = jnp.pad(pos_pad, ((0, 0), (0, 5)))
    posT = pos_pad.T

    r1 = lambda a: a.reshape(1, -1)
    f1, t1, s1 = _dense1(x_pad, W0, r1(b0), We1[:64], We1[64:], r1(be1))
    idx_a = _knn(pos_pad, posT, 0)
    ga = _sc_gather_reduce(t1, idx_a.reshape(-1), 64)
    idx_b = _knn(pos_pad, posT, 1)
    gb = _sc_gather_reduce(t1, idx_b.reshape(-1), 64)
    u1, v1, mx1, mn1 = [jnp.concatenate([ga[i], gb[i]]) for i in range(4)]
    S1 = _stats(s1, u1, v1, 64)
    f2, f3, t3, s3 = _dense2(S1, r1(g1), r1(bt1), f1, s1, mx1, mn1,
                             W2, r1(b2), We3[:128], We3[128:], r1(be3))
    g3a = _sc_gather_reduce(t3, idx_a.reshape(-1), 128)
    partial = _final_partial(f1, f2, f3,
                             Wout[:64], Wout[64:128], Wout[128:256],
                             r1(bout))
    g3b = _sc_gather_reduce(t3, idx_b.reshape(-1), 128)
    u3, v3, mx3, mn3 = [jnp.concatenate([g3a[i], g3b[i]]) for i in range(4)]
    S3 = _stats(s3, u3, v3, 128)
    out = _final2(S3, r1(g3), r1(bt3), f3, s3, mx3, mn3, partial,
                  Wout[256:384])
    return out[:N]


# speedup vs baseline: 1.0203x; 1.0203x over previous
import functools

import jax
import jax.numpy as jnp
from jax import lax
from jax.experimental import pallas as pl
from jax.experimental.pallas import tpu as pltpu
from jax.experimental.pallas import tpu_sc as plsc

N = 10000
K = 16
NP = 10240
ROW_BLK = 128
DENSE_BLK = 512
STAT_BLK = 400
EPS = 1e-5
NW = 32


def _knn_body(posr_ref, posT_ref, idx_ref):
    r0 = posr_ref[:, 0:1]
    r1 = posr_ref[:, 1:2]
    r2 = posr_ref[:, 2:3]
    c0 = posT_ref[0:1, :]
    c1 = posT_ref[1:2, :]
    c2 = posT_ref[2:3, :]
    d = -(((r0 - c0) ** 2 + (r1 - c1) ** 2) + (r2 - c2) ** 2)
    h = NP // 2
    dL = d[:, :h]
    dR = d[:, h:]
    jL = lax.broadcasted_iota(jnp.int32, (ROW_BLK, h), 1).astype(jnp.float32)
    jR = jL + float(h)
    win = dL >= dR
    val = jnp.where(win, dL, dR)
    vidx = jnp.where(win, jL, jR)
    lose = jnp.where(win, dR, dL)
    loseidx = jnp.where(win, jR, jL)
    bigf = jnp.float32(3.0e7)
    neginf = jnp.float32(-jnp.inf)
    cols = []
    m = jnp.max(val, axis=1, keepdims=True)
    for t in range(K):
        js = jnp.min(jnp.where(val == m, vidx, bigf), axis=1, keepdims=True)
        cols.append(js)
        jm = jnp.where(js >= h, js - float(h), js)
        smask = jL == jm
        val = jnp.where(smask, lose, val)
        vidx = jnp.where(smask, loseidx, vidx)
        lose = jnp.where(smask, neginf, lose)
        if t < K - 1:
            m = jnp.max(val, axis=1, keepdims=True)
    idx_ref[...] = jnp.concatenate(cols, axis=1).astype(jnp.int32)


def _knn(pos_pad, posT, half):
    nb = NP // ROW_BLK // 2
    return pl.pallas_call(
        _knn_body,
        grid=(nb,),
        in_specs=[
            pl.BlockSpec((ROW_BLK, 8), lambda i: (i + half * nb, 0)),
            pl.BlockSpec((8, NP), lambda i: (0, 0)),
        ],
        out_specs=pl.BlockSpec((ROW_BLK, K), lambda i: (i, 0)),
        out_shape=jax.ShapeDtypeStruct((NP // 2, K), jnp.int32),
    )(pos_pad, posT)


def _dense1_body(x_ref, W0_ref, b0_ref, Wa_ref, Wb_ref, be_ref,
                 f1_ref, t1_ref, s1_ref):
    x = x_ref[...]
    f1 = jnp.dot(x, W0_ref[...], preferred_element_type=jnp.float32) + b0_ref[...]
    f1_ref[...] = f1
    t1 = jnp.dot(f1, Wb_ref[...], preferred_element_type=jnp.float32)
    t1_ref[...] = jnp.concatenate([t1, jnp.zeros_like(t1)], axis=1)
    s1_ref[...] = (jnp.dot(f1, Wa_ref[...], preferred_element_type=jnp.float32)
                   - t1 + be_ref[...])


def _dense1(x_pad, W0, b0, Wa1, Wb1, be1):
    g = NP // DENSE_BLK
    blk = lambda r, c: pl.BlockSpec((r, c), lambda i: (i, 0))
    full = lambda r, c: pl.BlockSpec((r, c), lambda i: (0, 0))
    return pl.pallas_call(
        _dense1_body,
        grid=(g,),
        in_specs=[blk(DENSE_BLK, 64), full(64, 64), full(1, 64),
                  full(64, 64), full(64, 64), full(1, 64)],
        out_specs=[blk(DENSE_BLK, 64), blk(DENSE_BLK, 128), blk(DENSE_BLK, 64)],
        out_shape=[jax.ShapeDtypeStruct((NP, 64), jnp.float32),
                   jax.ShapeDtypeStruct((NP, 128), jnp.float32),
                   jax.ShapeDtypeStruct((NP, 64), jnp.float32)],
    )(x_pad, W0, b0, Wa1, Wb1, be1)


def _sc_gather_reduce(t_pad, idx_flat, C):
    G = 8
    W128 = 128
    NN = NP // 2
    npw = NN // NW
    nch = npw // G
    gk = G * K
    mesh = plsc.VectorSubcoreMesh(core_axis_name="c", subcore_axis_name="s")

    @functools.partial(
        pl.kernel,
        out_type=jax.ShapeDtypeStruct((4, NN, W128), jnp.float32),
        mesh=mesh,
        scratch_types=[
            pltpu.VMEM((npw * K,), jnp.int32),
            pltpu.VMEM((gk, W128), jnp.float32),
            pltpu.VMEM((4, G, W128), jnp.float32),
            pltpu.SemaphoreType.DMA,
        ],
    )
    def k(t_hbm, idx_hbm, out_hbm, idx_v, rows_v, stage_v, sem):
        wid = lax.axis_index("s") * 2 + lax.axis_index("c")
        base0 = wid * npw
        pltpu.sync_copy(idx_hbm.at[pl.ds(base0 * K, npw * K)], idx_v)

        def chunk(ch, carry):
            base = base0 + ch * G
            pltpu.async_copy(
                t_hbm.at[idx_v.at[pl.ds(ch * gk, gk)]], rows_v, sem).wait()
            for n in range(G):
                for cg in range(C // 16):
                    sl = pl.ds(cg * 16, 16)
                    r = rows_v[n * K, sl]
                    u = r
                    v = r * r
                    mx = r
                    mn = r
                    for kk in range(1, K):
                        r = rows_v[n * K + kk, sl]
                        u = u + r
                        v = v + r * r
                        mx = jnp.maximum(mx, r)
                        mn = jnp.minimum(mn, r)
                    stage_v[0, n, sl] = u
                    stage_v[1, n, sl] = v
                    stage_v[2, n, sl] = mx
                    stage_v[3, n, sl] = mn
            pltpu.sync_copy(stage_v, out_hbm.at[:, pl.ds(base, G), :])
            return carry

        lax.fori_loop(0, nch, chunk, 0)

    out = k(t_pad, idx_flat)
    return out[0], out[1], out[2], out[3]


def _stats_body(s_ref, u_ref, v_ref, o_ref):
    @pl.when(pl.program_id(0) == 0)
    def _():
        o_ref[...] = jnp.zeros_like(o_ref)

    s = s_ref[...]
    c = s.shape[1]
    u = u_ref[:, :c]
    v = v_ref[:, :c]
    acc = jnp.concatenate([
        jnp.sum(s, axis=0, keepdims=True),
        jnp.sum(s * s, axis=0, keepdims=True),
        jnp.sum(u, axis=0, keepdims=True),
        jnp.sum(s * u, axis=0, keepdims=True),
        jnp.sum(v, axis=0, keepdims=True),
        jnp.zeros((3, s.shape[1]), jnp.float32),
    ], axis=0)
    o_ref[...] += acc


def _stats(s, u, v, C):
    g = N // STAT_BLK
    blk = pl.BlockSpec((STAT_BLK, C), lambda i: (i, 0))
    blk128 = pl.BlockSpec((STAT_BLK, 128), lambda i: (i, 0))
    return pl.pallas_call(
        _stats_body,
        grid=(g,),
        in_specs=[blk, blk128, blk128],
        out_specs=pl.BlockSpec((8, C), lambda i: (0, 0)),
        out_shape=jax.ShapeDtypeStruct((8, C), jnp.float32),
    )(s, u, v)


def _bn_scale_shift(S_ref, g_ref, bt_ref):
    nk = float(N * K)
    S = S_ref[...]
    mu = (K * S[0:1, :] + S[2:3, :]) / nk
    ex2 = (K * S[1:2, :] + 2.0 * S[3:4, :] + S[4:5, :]) / nk
    var = ex2 - mu * mu
    scale = g_ref[...] / jnp.sqrt(var + EPS)
    shift = bt_ref[...] - mu * scale
    return scale, shift


def _dense2_body(S_ref, g_ref, bt_ref, f1_ref, s1_ref, mx_ref, mn_ref,
                 W2_ref, b2_ref, Wa3_ref, Wb3_ref, be3_ref,
                 f2_ref, f3_ref, t3_ref, s3_ref):
    scale, shift = _bn_scale_shift(S_ref, g_ref, bt_ref)
    sel = jnp.where(scale >= 0.0, mx_ref[:, :64], mn_ref[:, :64])
    f2 = jnp.maximum(scale * (s1_ref[...] + sel) + shift, 0.0) + f1_ref[...]
    f2_ref[...] = f2
    f3 = jnp.dot(f2, W2_ref[...], preferred_element_type=jnp.float32) + b2_ref[...]
    f3_ref[...] = f3
    t3 = jnp.dot(f3, Wb3_ref[...], preferred_element_type=jnp.float32)
    t3_ref[...] = t3
    s3_ref[...] = (jnp.dot(f3, Wa3_ref[...], preferred_element_type=jnp.float32)
                   - t3 + be3_ref[...])


def _dense2(S1, g1, bt1, f1, s1, mx1, mn1, W2, b2, Wa3, Wb3, be3):
    g = NP // DENSE_BLK
    blk = lambda r, c: pl.BlockSpec((r, c), lambda i: (i, 0))
    full = lambda r, c: pl.BlockSpec((r, c), lambda i: (0, 0))
    return pl.pallas_call(
        _dense2_body,
        grid=(g,),
        in_specs=[full(8, 64), full(1, 64), full(1, 64),
                  blk(DENSE_BLK, 64), blk(DENSE_BLK, 64),
                  blk(DENSE_BLK, 128), blk(DENSE_BLK, 128),
                  full(64, 128), full(1, 128),
                  full(128, 128), full(128, 128), full(1, 128)],
        out_specs=[blk(DENSE_BLK, 64), blk(DENSE_BLK, 128),
                   blk(DENSE_BLK, 128), blk(DENSE_BLK, 128)],
        out_shape=[jax.ShapeDtypeStruct((NP, 64), jnp.float32),
                   jax.ShapeDtypeStruct((NP, 128), jnp.float32),
                   jax.ShapeDtypeStruct((NP, 128), jnp.float32),
                   jax.ShapeDtypeStruct((NP, 128), jnp.float32)],
    )(S1, g1, bt1, f1, s1, mx1, mn1, W2, b2, Wa3, Wb3, be3)


def _final_partial_body(f1_ref, f2_ref, f3_ref,
                        Wo1_ref, Wo2_ref, Wo3_ref, bout_ref, o_ref):
    o = jnp.dot(f1_ref[...], Wo1_ref[...], preferred_element_type=jnp.float32)
    o = o + jnp.dot(f2_ref[...], Wo2_ref[...], preferred_element_type=jnp.float32)
    o = o + jnp.dot(f3_ref[...], Wo3_ref[...], preferred_element_type=jnp.float32)
    o_ref[...] = o + bout_ref[...]


def _final_partial(f1, f2, f3, Wo1, Wo2, Wo3, bout):
    g = NP // DENSE_BLK
    blk = lambda r, c: pl.BlockSpec((r, c), lambda i: (i, 0))
    full = lambda r, c: pl.BlockSpec((r, c), lambda i: (0, 0))
    return pl.pallas_call(
        _final_partial_body,
        grid=(g,),
        in_specs=[blk(DENSE_BLK, 64), blk(DENSE_BLK, 64),
                  blk(DENSE_BLK, 128),
                  full(64, 128), full(64, 128),
                  full(128, 128), full(1, 128)],
        out_specs=blk(DENSE_BLK, 128),
        out_shape=jax.ShapeDtypeStruct((NP, 128), jnp.float32),
    )(f1, f2, f3, Wo1, Wo2, Wo3, bout)


def _final2_body(S_ref, g_ref, bt_ref, f3_ref, s3_ref, mx_ref, mn_ref,
                 part_ref, Wo4_ref, o_ref):
    scale, shift = _bn_scale_shift(S_ref, g_ref, bt_ref)
    sel = jnp.where(scale >= 0.0, mx_ref[...], mn_ref[...])
    f4 = jnp.maximum(scale * (s3_ref[...] + sel) + shift, 0.0) + f3_ref[...]
    o_ref[...] = part_ref[...] + jnp.dot(
        f4, Wo4_ref[...], preferred_element_type=jnp.float32)


def _final2(S3, g3, bt3, f3, s3, mx3, mn3, partial, Wo4):
    g = NP // DENSE_BLK
    blk = lambda r, c: pl.BlockSpec((r, c), lambda i: (i, 0))
    full = lambda r, c: pl.BlockSpec((r, c), lambda i: (0, 0))
    return pl.pallas_call(
        _final2_body,
        grid=(g,),
        in_specs=[full(8, 128), full(1, 128), full(1, 128),
                  blk(DENSE_BLK, 128), blk(DENSE_BLK, 128),
                  blk(DENSE_BLK, 128), blk(DENSE_BLK, 128),
                  blk(DENSE_BLK, 128), full(128, 128)],
        out_specs=blk(DENSE_BLK, 128),
        out_shape=jax.ShapeDtypeStruct((NP, 128), jnp.float32),
    )(S3, g3, bt3, f3, s3, mx3, mn3, partial, Wo4)


def kernel(point_features, point_coords, W0, b0, We1, be1, g1, bt1,
           W2, b2, We3, be3, g3, bt3, Wout, bout):
    pos = point_coords[:, 1:4]
    x = jnp.concatenate([pos, point_features], axis=1)
    x_pad = jnp.pad(x, ((0, NP - N), (0, 0)))
    pos_pad = jnp.pad(pos, ((0, NP - N), (0, 0)), constant_values=1e6)
    pos_pad = jnp.pad(pos_pad, ((0, 0), (0, 5)))
    posT = pos_pad.T

    r1 = lambda a: a.reshape(1, -1)
    f1, t1, s1 = _dense1(x_pad, W0, r1(b0), We1[:64], We1[64:], r1(be1))
    idx_a = _knn(pos_pad, posT, 0)
    ga = _sc_gather_reduce(t1, idx_a.reshape(-1), 64)
    idx_b = _knn(pos_pad, posT, 1)
    gb = _sc_gather_reduce(t1, idx_b.reshape(-1), 64)
    u1, v1, mx1, mn1 = [jnp.concatenate([ga[i], gb[i]]) for i in range(4)]
    S1 = _stats(s1, u1, v1, 64)
    f2, f3, t3, s3 = _dense2(S1, r1(g1), r1(bt1), f1, s1, mx1, mn1,
                             W2, r1(b2), We3[:128], We3[128:], r1(be3))
    g3a = _sc_gather_reduce(t3, idx_a.reshape(-1), 128)
    partial = _final_partial(f1, f2, f3,
                             Wout[:64], Wout[64:128], Wout[128:256],
                             r1(bout))
    g3b = _sc_gather_reduce(t3, idx_b.reshape(-1), 128)
    u3, v3, mx3, mn3 = [jnp.concatenate([g3a[i], g3b[i]]) for i in range(4)]
    S3 = _stats(s3, u3, v3, 128)
    out = _final2(S3, r1(g3), r1(bt3), f3, s3, mx3, mn3, partial,
                  Wout[256:384])
    return out[:N]
